# baseline (speedup 1.0000x reference)
import jax
import jax.numpy as jnp
from jax import lax
from jax.experimental import pallas as pl
from jax.experimental.pallas import tpu as pltpu
from jax.experimental.pallas import tpu_sc as plsc

B = 16384
M = 20
D = 32
NC = 2
NS = 16
NW = NC * NS
QPW = B // NW
C = 64
NCHUNK = QPW // C
IPC = C * M
GW = 128
NSUB = IPC // GW


V = 1000000
NBLK = 244


def _relayout_body(t_hbm, tail_hbm, out_hbm, dbuf, out_buf, sem):
    wid = lax.axis_index("s") * NC + lax.axis_index("c")
    iota = lax.iota(jnp.int32, 16)

    def do_block(j):
        off = pl.multiple_of(j * 128, 128)
        pltpu.sync_copy(t_hbm.at[:, pl.ds(off, 128)], dbuf)
        for v in range(128):
            lv = jnp.full((16,), v, jnp.int32)
            g0 = plsc.load_gather(dbuf, [iota, lv])
            g1 = plsc.load_gather(dbuf, [iota + 16, lv])
            out_buf[pl.ds(v * D, 16)] = g0
            out_buf[pl.ds(v * D + 16, 16)] = g1
        pltpu.sync_copy(out_buf, out_hbm.at[pl.ds(j * 128 * D, 128 * D)])

    def blk(i, carry):
        do_block(wid + NW * i)
        return carry

    lax.fori_loop(0, NBLK, blk, 0)
    @pl.when(wid < 4)
    def _():
        do_block(7808 + wid)

    @pl.when(wid == 4)
    def _():
        pltpu.sync_copy(tail_hbm, out_buf.at[pl.ds(0, 64 * D)])
        pltpu.sync_copy(out_buf.at[pl.ds(0, 64 * D)],
                        out_hbm.at[pl.ds((V - 64) * D, 64 * D)])


def _relayout(table):
    t = jnp.swapaxes(table, 0, 1)
    tail = table[V - 64:].reshape(64 * D)
    mesh = plsc.VectorSubcoreMesh(
        core_axis_name="c", subcore_axis_name="s",
        num_cores=NC, num_subcores=NS)
    run = pl.kernel(
        _relayout_body,
        out_type=jax.ShapeDtypeStruct((V * D,), jnp.float32),
        mesh=mesh,
        compiler_params=pltpu.CompilerParams(
            needs_layout_passes=False, use_tc_tiling_on_sc=True),
        scratch_types=[
            pltpu.VMEM((D, 128), jnp.float32),
            pltpu.VMEM((128 * D,), jnp.float32),
            pltpu.SemaphoreType.DMA,
        ],
    )
    return run(t, tail).reshape(V, D)


def _ges_body(qri_hbm, qrc_hbm, qrb_hbm, match_hbm, out_t, out_hbm,
              mi_idx, id_rows, cat_rows, br_rows, m_rows, logits, isem, sem):
    wid = lax.axis_index("s") * NC + lax.axis_index("c")

    def chunk_body(c, carry):
        b0 = wid * QPW + c * C
        idescs = [
            pltpu.async_copy(qri_hbm.at[pl.ds(b0, C)], id_rows, isem),
            pltpu.async_copy(qrc_hbm.at[pl.ds(b0, C)], cat_rows, isem),
            pltpu.async_copy(qrb_hbm.at[pl.ds(b0, C)], br_rows, isem),
        ]
        for j in range(NSUB):
            idescs.append(pltpu.async_copy(
                match_hbm.at[pl.ds(b0 * M + j * GW, GW)], mi_idx.at[j], isem))
        for d_ in idescs:
            d_.wait()
        descs = []
        for j in range(NSUB):
            descs.append(pltpu.async_copy(
                out_t.at[mi_idx.at[j]], m_rows.at[pl.ds(j * GW, GW)], sem))
        for d_ in descs:
            d_.wait()

        third = jnp.float32(1.0 / 3.0)
        lane = lax.iota(jnp.int32, 16)

        def g_body(g, carry_q):
            accs = [jnp.zeros((16,), jnp.float32) for _ in range(5)]
            for bi in range(4):
                b = g * 4 + bi
                h0 = (id_rows[b, pl.ds(0, 16)] + cat_rows[b, pl.ds(0, 16)]
                      + br_rows[b, pl.ds(0, 16)]) * third
                h1 = (id_rows[b, pl.ds(16, 16)] + cat_rows[b, pl.ds(16, 16)]
                      + br_rows[b, pl.ds(16, 16)]) * third
                for m in range(M):
                    row = b * M + m
                    p = (m_rows[row, pl.ds(0, 16)] * h0
                         + m_rows[row, pl.ds(16, 16)] * h1)
                    s = jnp.sum(p)
                    k, ln = divmod(bi * M + m, 16)
                    accs[k] = jnp.where(lane == ln, s, accs[k])
            for k in range(5):
                logits[pl.ds(g * 80 + k * 16, 16)] = accs[k]
            return carry_q

        lax.fori_loop(0, C // 4, g_body, 0)
        pltpu.sync_copy(logits, out_hbm.at[pl.ds(b0 * M, IPC)])
        return carry

    lax.fori_loop(0, NCHUNK, chunk_body, 0)


def kernel(query_item_id, query_cat_id, query_brand_id, match,
           id_table, cat_table, brand_table, out_table):
    qid = query_item_id.reshape(B).astype(jnp.int32)
    qcat = query_cat_id.reshape(B).astype(jnp.int32)
    qbrand = query_brand_id.reshape(B).astype(jnp.int32)
    match_r = match.reshape(B * M).astype(jnp.int32)
    qrows_i = jnp.take(id_table, qid, axis=0)
    qrows_c = jnp.take(cat_table, qcat, axis=0)
    qrows_b = jnp.take(brand_table, qbrand, axis=0)

    mesh = plsc.VectorSubcoreMesh(
        core_axis_name="c", subcore_axis_name="s",
        num_cores=NC, num_subcores=NS)
    run = pl.kernel(
        _ges_body,
        out_type=jax.ShapeDtypeStruct((B * M,), jnp.float32),
        mesh=mesh,
        compiler_params=pltpu.CompilerParams(
            needs_layout_passes=False, use_tc_tiling_on_sc=False),
        scratch_types=[
            pltpu.VMEM((NSUB, GW), jnp.int32),
            pltpu.VMEM((C, D), jnp.float32),
            pltpu.VMEM((C, D), jnp.float32),
            pltpu.VMEM((C, D), jnp.float32),
            pltpu.VMEM((IPC, D), jnp.float32),
            pltpu.VMEM((IPC,), jnp.float32),
            pltpu.SemaphoreType.DMA,
            pltpu.SemaphoreType.DMA,
        ],
    )
    flat = run(qrows_i, qrows_c, qrows_b, match_r, _relayout(out_table))
    return flat.reshape(B, M)


# speedup vs baseline: 1.3087x; 1.3087x over previous
import jax
import jax.numpy as jnp
from jax import lax
from jax.experimental import pallas as pl
from jax.experimental.pallas import tpu as pltpu
from jax.experimental.pallas import tpu_sc as plsc

B = 16384
M = 20
D = 32
NC = 2
NS = 16
NW = NC * NS
QPW = B // NW
C = 64
NCHUNK = QPW // C
IPC = C * M
GW = 128
NSUB = IPC // GW


V = 1000000
NBLK = 244


def _relayout_body(t_hbm, tail_hbm, out_hbm,
                   dbuf0, dbuf1, obuf0, obuf1, isem0, isem1, osem0, osem1):
    wid = lax.axis_index("s") * NC + lax.axis_index("c")
    iota = lax.iota(jnp.int32, 16)

    def in_slice(j):
        off = pl.multiple_of(j * 128, 128)
        return t_hbm.at[:, pl.ds(off, 128)]

    def out_slice(j):
        return out_hbm.at[pl.ds(j * 128 * D, 128 * D)]

    def transpose_block(dbuf, obuf, nv=128):
        for v in range(nv):
            lv = jnp.full((16,), v, jnp.int32)
            g0 = plsc.load_gather(dbuf, [iota, lv])
            g1 = plsc.load_gather(dbuf, [iota + 16, lv])
            obuf[pl.ds(v * D, 16)] = g0
            obuf[pl.ds(v * D + 16, 16)] = g1

    def jat(i):
        return wid + NW * i

    pltpu.async_copy(in_slice(jat(0)), dbuf0, isem0)

    def round_body(o, carry):
        j0 = jat(2 * o)
        j1 = jat(2 * o + 1)
        pltpu.make_async_copy(in_slice(j0), dbuf0, isem0).wait()
        pltpu.async_copy(in_slice(j1), dbuf1, isem1)

        @pl.when(o > 0)
        def _():
            pltpu.make_async_copy(obuf0, out_slice(j0), osem0).wait()

        transpose_block(dbuf0, obuf0)
        pltpu.async_copy(obuf0, out_slice(j0), osem0)
        pltpu.make_async_copy(in_slice(j1), dbuf1, isem1).wait()

        @pl.when(o < NBLK // 2 - 1)
        def _():
            pltpu.async_copy(in_slice(jat(2 * o + 2)), dbuf0, isem0)

        @pl.when(o > 0)
        def _():
            pltpu.make_async_copy(obuf1, out_slice(j1), osem1).wait()

        transpose_block(dbuf1, obuf1)
        pltpu.async_copy(obuf1, out_slice(j1), osem1)
        return carry

    lax.fori_loop(0, NBLK // 2, round_body, 0)
    pltpu.make_async_copy(obuf0, out_slice(jat(NBLK - 2)), osem0).wait()
    pltpu.make_async_copy(obuf1, out_slice(jat(NBLK - 1)), osem1).wait()

    @pl.when(wid < 4)
    def _():
        j = 7808 + wid
        pltpu.sync_copy(in_slice(j), dbuf0)
        transpose_block(dbuf0, obuf0)
        pltpu.sync_copy(obuf0, out_slice(j))

    @pl.when(wid == 4)
    def _():
        pltpu.sync_copy(tail_hbm, obuf0.at[pl.ds(0, 64 * D)])
        pltpu.sync_copy(obuf0.at[pl.ds(0, 64 * D)],
                        out_hbm.at[pl.ds((V - 64) * D, 64 * D)])


def _relayout(table):
    t = jnp.swapaxes(table, 0, 1)
    tail = table[V - 64:].reshape(64 * D)
    mesh = plsc.VectorSubcoreMesh(
        core_axis_name="c", subcore_axis_name="s",
        num_cores=NC, num_subcores=NS)
    run = pl.kernel(
        _relayout_body,
        out_type=jax.ShapeDtypeStruct((V * D,), jnp.float32),
        mesh=mesh,
        compiler_params=pltpu.CompilerParams(
            needs_layout_passes=False, use_tc_tiling_on_sc=True),
        scratch_types=[
            pltpu.VMEM((D, 128), jnp.float32),
            pltpu.VMEM((D, 128), jnp.float32),
            pltpu.VMEM((128 * D,), jnp.float32),
            pltpu.VMEM((128 * D,), jnp.float32),
            pltpu.SemaphoreType.DMA,
            pltpu.SemaphoreType.DMA,
            pltpu.SemaphoreType.DMA,
            pltpu.SemaphoreType.DMA,
        ],
    )
    return run(t, tail).reshape(V, D)


def _ges_body(qri_hbm, qrc_hbm, qrb_hbm, match_hbm, out_t, out_hbm,
              mi_idx, id_rows, cat_rows, br_rows, m_rows, logits, isem, sem):
    wid = lax.axis_index("s") * NC + lax.axis_index("c")

    def chunk_body(c, carry):
        b0 = wid * QPW + c * C
        idescs = [
            pltpu.async_copy(qri_hbm.at[pl.ds(b0, C)], id_rows, isem),
            pltpu.async_copy(qrc_hbm.at[pl.ds(b0, C)], cat_rows, isem),
            pltpu.async_copy(qrb_hbm.at[pl.ds(b0, C)], br_rows, isem),
        ]
        for j in range(NSUB):
            idescs.append(pltpu.async_copy(
                match_hbm.at[pl.ds(b0 * M + j * GW, GW)], mi_idx.at[j], isem))
        for d_ in idescs:
            d_.wait()
        descs = []
        for j in range(NSUB):
            descs.append(pltpu.async_copy(
                out_t.at[mi_idx.at[j]], m_rows.at[pl.ds(j * GW, GW)], sem))
        for d_ in descs:
            d_.wait()

        third = jnp.float32(1.0 / 3.0)
        lane = lax.iota(jnp.int32, 16)

        def g_body(g, carry_q):
            accs = [jnp.zeros((16,), jnp.float32) for _ in range(5)]
            for bi in range(4):
                b = g * 4 + bi
                h0 = (id_rows[b, pl.ds(0, 16)] + cat_rows[b, pl.ds(0, 16)]
                      + br_rows[b, pl.ds(0, 16)]) * third
                h1 = (id_rows[b, pl.ds(16, 16)] + cat_rows[b, pl.ds(16, 16)]
                      + br_rows[b, pl.ds(16, 16)]) * third
                for m in range(M):
                    row = b * M + m
                    p = (m_rows[row, pl.ds(0, 16)] * h0
                         + m_rows[row, pl.ds(16, 16)] * h1)
                    s = jnp.sum(p)
                    k, ln = divmod(bi * M + m, 16)
                    accs[k] = jnp.where(lane == ln, s, accs[k])
            for k in range(5):
                logits[pl.ds(g * 80 + k * 16, 16)] = accs[k]
            return carry_q

        lax.fori_loop(0, C // 4, g_body, 0)
        pltpu.sync_copy(logits, out_hbm.at[pl.ds(b0 * M, IPC)])
        return carry

    lax.fori_loop(0, NCHUNK, chunk_body, 0)


def kernel(query_item_id, query_cat_id, query_brand_id, match,
           id_table, cat_table, brand_table, out_table):
    qid = query_item_id.reshape(B).astype(jnp.int32)
    qcat = query_cat_id.reshape(B).astype(jnp.int32)
    qbrand = query_brand_id.reshape(B).astype(jnp.int32)
    match_r = match.reshape(B * M).astype(jnp.int32)
    qrows_i = jnp.take(id_table, qid, axis=0)
    qrows_c = jnp.take(cat_table, qcat, axis=0)
    qrows_b = jnp.take(brand_table, qbrand, axis=0)

    mesh = plsc.VectorSubcoreMesh(
        core_axis_name="c", subcore_axis_name="s",
        num_cores=NC, num_subcores=NS)
    run = pl.kernel(
        _ges_body,
        out_type=jax.ShapeDtypeStruct((B * M,), jnp.float32),
        mesh=mesh,
        compiler_params=pltpu.CompilerParams(
            needs_layout_passes=False, use_tc_tiling_on_sc=False),
        scratch_types=[
            pltpu.VMEM((NSUB, GW), jnp.int32),
            pltpu.VMEM((C, D), jnp.float32),
            pltpu.VMEM((C, D), jnp.float32),
            pltpu.VMEM((C, D), jnp.float32),
            pltpu.VMEM((IPC, D), jnp.float32),
            pltpu.VMEM((IPC,), jnp.float32),
            pltpu.SemaphoreType.DMA,
            pltpu.SemaphoreType.DMA,
        ],
    )
    flat = run(qrows_i, qrows_c, qrows_b, match_r, _relayout(out_table))
    return flat.reshape(B, M)


# speedup vs baseline: 1.7003x; 1.2993x over previous
import jax
import jax.numpy as jnp
from jax import lax
from jax.experimental import pallas as pl
from jax.experimental.pallas import tpu as pltpu
from jax.experimental.pallas import tpu_sc as plsc

B = 16384
M = 20
D = 32
NC = 2
NS = 16
NW = NC * NS
QPW = B // NW
C = 64
NCHUNK = QPW // C
IPC = C * M
GW = 128
NSUB = IPC // GW
NG = C // 4


def _ges_body(qri_hbm, qrc_hbm, qrb_hbm, match_hbm, out_t, out_hbm,
              mi0, mi1, idr0, idr1, ctr0, ctr1, brr0, brr1,
              mr0, mr1, lg0, lg1, isem0, isem1, gsem0, gsem1,
              osem0, osem1):
    wid = lax.axis_index("s") * NC + lax.axis_index("c")
    third = jnp.float32(1.0 / 3.0)
    lane = lax.iota(jnp.int32, 16)

    def base(c):
        return wid * QPW + c * C

    def fire_idx(c, mi, idr, ctr, brr, isem):
        b0 = base(c)
        pltpu.async_copy(qri_hbm.at[pl.ds(b0, C)], idr, isem)
        pltpu.async_copy(qrc_hbm.at[pl.ds(b0, C)], ctr, isem)
        pltpu.async_copy(qrb_hbm.at[pl.ds(b0, C)], brr, isem)
        for j in range(NSUB):
            pltpu.async_copy(
                match_hbm.at[pl.ds(b0 * M + j * GW, GW)], mi.at[j], isem)

    def drain_idx(mi, idr, ctr, brr, isem):
        pltpu.make_async_copy(qri_hbm.at[pl.ds(0, C)], idr, isem).wait()
        pltpu.make_async_copy(qrc_hbm.at[pl.ds(0, C)], ctr, isem).wait()
        pltpu.make_async_copy(qrb_hbm.at[pl.ds(0, C)], brr, isem).wait()
        for j in range(NSUB):
            pltpu.make_async_copy(
                match_hbm.at[pl.ds(j * GW, GW)], mi.at[j], isem).wait()

    def fire_gathers(mi, mr, gsem):
        for j in range(NSUB):
            pltpu.async_copy(
                out_t.at[mi.at[j]], mr.at[pl.ds(j * GW, GW)], gsem)

    def drain_gathers(mi, mr, gsem):
        for j in range(NSUB):
            pltpu.make_async_copy(
                out_t.at[mi.at[j]], mr.at[pl.ds(j * GW, GW)], gsem).wait()

    def compute_half(idr, ctr, brr, mr, lg, glo, ghi):
        def g_body(g, carry_q):
            accs = [jnp.zeros((16,), jnp.float32) for _ in range(5)]
            for bi in range(4):
                b = g * 4 + bi
                h0 = (idr[b, pl.ds(0, 16)] + ctr[b, pl.ds(0, 16)]
                      + brr[b, pl.ds(0, 16)]) * third
                h1 = (idr[b, pl.ds(16, 16)] + ctr[b, pl.ds(16, 16)]
                      + brr[b, pl.ds(16, 16)]) * third
                for m in range(M):
                    row = b * M + m
                    p = (mr[row, pl.ds(0, 16)] * h0
                         + mr[row, pl.ds(16, 16)] * h1)
                    s = jnp.sum(p)
                    k, ln = divmod(bi * M + m, 16)
                    accs[k] = jnp.where(lane == ln, s, accs[k])
            for k in range(5):
                lg[pl.ds(g * 80 + k * 16, 16)] = accs[k]
            return carry_q

        lax.fori_loop(glo, ghi, g_body, 0)

    def fire_out(c, lg, osem):
        pltpu.async_copy(lg, out_hbm.at[pl.ds(base(c) * M, IPC)], osem)

    def drain_out(lg, osem):
        pltpu.make_async_copy(lg, out_hbm.at[pl.ds(0, IPC)], osem).wait()

    s0 = (mi0, idr0, ctr0, brr0)
    s1 = (mi1, idr1, ctr1, brr1)

    fire_idx(0, *s0, isem0)
    drain_idx(*s0, isem0)
    fire_gathers(mi0, mr0, gsem0)

    def round_body(r, carry):
        c0 = 2 * r
        c1 = 2 * r + 1
        drain_gathers(mi0, mr0, gsem0)
        fire_idx(c1, *s1, isem1)
        compute_half(idr0, ctr0, brr0, mr0, lg0, 0, NG // 2)
        drain_idx(*s1, isem1)
        fire_gathers(mi1, mr1, gsem1)
        compute_half(idr0, ctr0, brr0, mr0, lg0, NG // 2, NG)

        @pl.when(r > 0)
        def _():
            drain_out(lg0, osem0)

        fire_out(c0, lg0, osem0)

        drain_gathers(mi1, mr1, gsem1)

        @pl.when(r < NCHUNK // 2 - 1)
        def _():
            fire_idx(c1 + 1, *s0, isem0)

        compute_half(idr1, ctr1, brr1, mr1, lg1, 0, NG // 2)

        @pl.when(r < NCHUNK // 2 - 1)
        def _():
            drain_idx(*s0, isem0)
            fire_gathers(mi0, mr0, gsem0)

        compute_half(idr1, ctr1, brr1, mr1, lg1, NG // 2, NG)

        @pl.when(r > 0)
        def _():
            drain_out(lg1, osem1)

        fire_out(c1, lg1, osem1)
        return carry

    lax.fori_loop(0, NCHUNK // 2, round_body, 0)
    drain_out(lg0, osem0)
    drain_out(lg1, osem1)


def kernel(query_item_id, query_cat_id, query_brand_id, match,
           id_table, cat_table, brand_table, out_table):
    qid = query_item_id.reshape(B).astype(jnp.int32)
    qcat = query_cat_id.reshape(B).astype(jnp.int32)
    qbrand = query_brand_id.reshape(B).astype(jnp.int32)
    match_r = match.reshape(B * M).astype(jnp.int32)
    qrows_i = jnp.take(id_table, qid, axis=0)
    qrows_c = jnp.take(cat_table, qcat, axis=0)
    qrows_b = jnp.take(brand_table, qbrand, axis=0)

    mesh = plsc.VectorSubcoreMesh(
        core_axis_name="c", subcore_axis_name="s",
        num_cores=NC, num_subcores=NS)
    run = pl.kernel(
        _ges_body,
        out_type=jax.ShapeDtypeStruct((B * M,), jnp.float32),
        mesh=mesh,
        compiler_params=pltpu.CompilerParams(
            needs_layout_passes=False, use_tc_tiling_on_sc=False),
        scratch_types=[
            pltpu.VMEM((NSUB, GW), jnp.int32),
            pltpu.VMEM((NSUB, GW), jnp.int32),
            pltpu.VMEM((C, D), jnp.float32),
            pltpu.VMEM((C, D), jnp.float32),
            pltpu.VMEM((C, D), jnp.float32),
            pltpu.VMEM((C, D), jnp.float32),
            pltpu.VMEM((C, D), jnp.float32),
            pltpu.VMEM((C, D), jnp.float32),
            pltpu.VMEM((IPC, D), jnp.float32),
            pltpu.VMEM((IPC, D), jnp.float32),
            pltpu.VMEM((IPC,), jnp.float32),
            pltpu.VMEM((IPC,), jnp.float32),
            pltpu.SemaphoreType.DMA,
            pltpu.SemaphoreType.DMA,
            pltpu.SemaphoreType.DMA,
            pltpu.SemaphoreType.DMA,
            pltpu.SemaphoreType.DMA,
            pltpu.SemaphoreType.DMA,
        ],
    )
    flat = run(qrows_i, qrows_c, qrows_b, match_r, out_table)
    return flat.reshape(B, M)
